# Initial kernel scaffold; baseline (speedup 1.0000x reference)
import jax
import jax.numpy as jnp
from jax.experimental import pallas as pl


def kernel(x, preproc):
    raise NotImplementedError("write your pallas kernel here")


# speedup vs baseline: 2.0466x; 2.0466x over previous
import numpy as np
import jax
import jax.numpy as jnp
from jax.experimental import pallas as pl
from jax.experimental.pallas import tpu as pltpu

_N, _S, _F = 4096, 128, 96
_BN = 32

_RUNS22 = ((6, 18), (21, 33), (36, 48), (51, 60), (63, 69), (75, 84), (87, 93))
_IDX2212 = ([0], [1, 2, 3], [4], [5, 6, 7], [8, 9], [10, 11], [12], [13],
            [14, 15, 16], [17], [18], [19, 20, 21])
_IDX127 = ([0, 1], [2, 3], [4, 5], [6, 7], [7, 8], [9, 10], [10, 11])
_IDX74 = ([0, 2], [1, 2], [3, 4], [5, 6])


def _group_mean(x, groups):
    pieces = []
    for idx in groups:
        seg = x[:, 3 * idx[0]:3 * idx[0] + 3, :]
        for j in idx[1:]:
            seg = seg + x[:, 3 * j:3 * j + 3, :]
        if len(idx) > 1:
            seg = seg * (1.0 / len(idx))
        pieces.append(seg)
    return jnp.concatenate(pieces, axis=1)


def _body(p_ref, x_ref, o32, o22, o12, o7, o4):
    x = x_ref[...] * p_ref[0]
    xt = jnp.transpose(x, (0, 2, 1))
    o32[...] = xt
    x22 = jnp.concatenate([xt[:, a:b, :] for a, b in _RUNS22], axis=1)
    o22[...] = x22
    x12 = _group_mean(x22, _IDX2212)
    o12[...] = x12
    x7 = _group_mean(x12, _IDX127)
    o7[...] = x7
    x4 = _group_mean(x7, _IDX74)
    o4[...] = x4


def _out_spec(d):
    return pl.BlockSpec((_BN, d, _S), lambda i: (i, 0, 0))


@jax.jit
def kernel(x, preproc):
    p = jnp.asarray(preproc, jnp.float32).reshape((1,))
    grid = (_N // _BN,)
    f32 = jnp.float32
    out = pl.pallas_call(
        _body,
        grid=grid,
        in_specs=[
            pl.BlockSpec(memory_space=pltpu.SMEM),
            pl.BlockSpec((_BN, _S, _F), lambda i: (i, 0, 0)),
        ],
        out_specs=[_out_spec(96), _out_spec(66), _out_spec(36), _out_spec(21),
                   _out_spec(12)],
        out_shape=[
            jax.ShapeDtypeStruct((_N, 96, _S), f32),
            jax.ShapeDtypeStruct((_N, 66, _S), f32),
            jax.ShapeDtypeStruct((_N, 36, _S), f32),
            jax.ShapeDtypeStruct((_N, 21, _S), f32),
            jax.ShapeDtypeStruct((_N, 12, _S), f32),
        ],
    )(p, x)
    return tuple(out)


# speedup vs baseline: 2.0964x; 1.0243x over previous
import numpy as np
import jax
import jax.numpy as jnp
from jax.experimental import pallas as pl
from jax.experimental.pallas import tpu as pltpu

_N, _S, _F = 4096, 128, 96
_BN = 64

_RUNS22 = ((6, 18), (21, 33), (36, 48), (51, 60), (63, 69), (75, 84), (87, 93))
_IDX2212 = ([0], [1, 2, 3], [4], [5, 6, 7], [8, 9], [10, 11], [12], [13],
            [14, 15, 16], [17], [18], [19, 20, 21])
_IDX127 = ([0, 1], [2, 3], [4, 5], [6, 7], [7, 8], [9, 10], [10, 11])
_IDX74 = ([0, 2], [1, 2], [3, 4], [5, 6])


def _group_mean(x, groups):
    pieces = []
    for idx in groups:
        seg = x[:, 3 * idx[0]:3 * idx[0] + 3, :]
        for j in idx[1:]:
            seg = seg + x[:, 3 * j:3 * j + 3, :]
        if len(idx) > 1:
            seg = seg * (1.0 / len(idx))
        pieces.append(seg)
    return jnp.concatenate(pieces, axis=1)


def _body(p_ref, x_ref, o32, o22, o12, o7, o4):
    x = x_ref[...] * p_ref[0]
    xt = jnp.transpose(x, (0, 2, 1))
    o32[...] = xt
    x22 = jnp.concatenate([xt[:, a:b, :] for a, b in _RUNS22], axis=1)
    o22[...] = x22
    x12 = _group_mean(x22, _IDX2212)
    o12[...] = x12
    x7 = _group_mean(x12, _IDX127)
    o7[...] = x7
    x4 = _group_mean(x7, _IDX74)
    o4[...] = x4


def _out_spec(d):
    return pl.BlockSpec((_BN, d, _S), lambda i: (i, 0, 0))


@jax.jit
def kernel(x, preproc):
    p = jnp.asarray(preproc, jnp.float32).reshape((1,))
    grid = (_N // _BN,)
    f32 = jnp.float32
    out = pl.pallas_call(
        _body,
        grid=grid,
        in_specs=[
            pl.BlockSpec(memory_space=pltpu.SMEM),
            pl.BlockSpec((_BN, _S, _F), lambda i: (i, 0, 0)),
        ],
        out_specs=[_out_spec(96), _out_spec(66), _out_spec(36), _out_spec(21),
                   _out_spec(12)],
        out_shape=[
            jax.ShapeDtypeStruct((_N, 96, _S), f32),
            jax.ShapeDtypeStruct((_N, 66, _S), f32),
            jax.ShapeDtypeStruct((_N, 36, _S), f32),
            jax.ShapeDtypeStruct((_N, 21, _S), f32),
            jax.ShapeDtypeStruct((_N, 12, _S), f32),
        ],
    )(p, x)
    return tuple(out)


# speedup vs baseline: 6.2930x; 3.0018x over previous
import numpy as np
import jax
import jax.numpy as jnp
from jax.experimental import pallas as pl
from jax.experimental.pallas import tpu as pltpu

_N, _S, _F = 4096, 128, 96
_BN = 64

_RUNS22 = ((6, 18), (21, 33), (36, 48), (51, 60), (63, 69), (75, 84), (87, 93))
_IDX2212 = ([0], [1, 2, 3], [4], [5, 6, 7], [8, 9], [10, 11], [12], [13],
            [14, 15, 16], [17], [18], [19, 20, 21])
_IDX127 = ([0, 1], [2, 3], [4, 5], [6, 7], [7, 8], [9, 10], [10, 11])
_IDX74 = ([0, 2], [1, 2], [3, 4], [5, 6])


def _group_mean(x, groups):
    pieces = []
    for idx in groups:
        seg = x[3 * idx[0]:3 * idx[0] + 3]
        for j in idx[1:]:
            seg = seg + x[3 * j:3 * j + 3]
        if len(idx) > 1:
            seg = seg * (1.0 / len(idx))
        pieces.append(seg)
    return jnp.concatenate(pieces, axis=0)


def _body(p_ref, x_ref, o32, o22, o12, o7, o4):
    xs = x_ref[...] * p_ref[0]
    o32[...] = xs
    xt = jnp.transpose(xs, (1, 0, 2))
    x22 = jnp.concatenate([xt[a:b] for a, b in _RUNS22], axis=0)
    o22[...] = x22
    x12 = _group_mean(x22, _IDX2212)
    o12[...] = x12
    x7 = _group_mean(x12, _IDX127)
    o7[...] = x7
    x4 = _group_mean(x7, _IDX74)
    o4[...] = x4


def _dmajor_spec(d):
    return pl.BlockSpec((d, _BN, _S), lambda i: (0, i, 0))


@jax.jit
def kernel(x, preproc):
    p = jnp.asarray(preproc, jnp.float32).reshape((1,))
    xt = jnp.transpose(x, (0, 2, 1))
    f32 = jnp.float32
    out = pl.pallas_call(
        _body,
        grid=(_N // _BN,),
        in_specs=[
            pl.BlockSpec(memory_space=pltpu.SMEM),
            pl.BlockSpec((_BN, _F, _S), lambda i: (i, 0, 0)),
        ],
        out_specs=[
            pl.BlockSpec((_BN, _F, _S), lambda i: (i, 0, 0)),
            _dmajor_spec(66), _dmajor_spec(36), _dmajor_spec(21),
            _dmajor_spec(12),
        ],
        out_shape=[
            jax.ShapeDtypeStruct((_N, 96, _S), f32),
            jax.ShapeDtypeStruct((66, _N, _S), f32),
            jax.ShapeDtypeStruct((36, _N, _S), f32),
            jax.ShapeDtypeStruct((21, _N, _S), f32),
            jax.ShapeDtypeStruct((12, _N, _S), f32),
        ],
    )(p, xt)
    x32 = out[0]
    rest = tuple(jnp.transpose(o, (1, 0, 2)) for o in out[1:])
    return (x32,) + rest


# speedup vs baseline: 6.4699x; 1.0281x over previous
import numpy as np
import jax
import jax.numpy as jnp
from jax.experimental import pallas as pl
from jax.experimental.pallas import tpu as pltpu

_N, _S, _F = 4096, 128, 96
_BN = 128

_RUNS22 = ((6, 18), (21, 33), (36, 48), (51, 60), (63, 69), (75, 84), (87, 93))
_IDX2212 = ([0], [1, 2, 3], [4], [5, 6, 7], [8, 9], [10, 11], [12], [13],
            [14, 15, 16], [17], [18], [19, 20, 21])
_IDX127 = ([0, 1], [2, 3], [4, 5], [6, 7], [7, 8], [9, 10], [10, 11])
_IDX74 = ([0, 2], [1, 2], [3, 4], [5, 6])


def _group_mean(x, groups):
    pieces = []
    for idx in groups:
        seg = x[3 * idx[0]:3 * idx[0] + 3]
        for j in idx[1:]:
            seg = seg + x[3 * j:3 * j + 3]
        if len(idx) > 1:
            seg = seg * (1.0 / len(idx))
        pieces.append(seg)
    return jnp.concatenate(pieces, axis=0)


def _body(p_ref, x_ref, o32, o22, o12, o7, o4):
    xs = x_ref[...] * p_ref[0]
    o32[...] = xs
    xt = jnp.transpose(xs, (1, 0, 2))
    x22 = jnp.concatenate([xt[a:b] for a, b in _RUNS22], axis=0)
    o22[...] = x22
    x12 = _group_mean(x22, _IDX2212)
    o12[...] = x12
    x7 = _group_mean(x12, _IDX127)
    o7[...] = x7
    x4 = _group_mean(x7, _IDX74)
    o4[...] = x4


def _dmajor_spec(d):
    return pl.BlockSpec((d, _BN, _S), lambda i: (0, i, 0))


@jax.jit
def kernel(x, preproc):
    p = jnp.asarray(preproc, jnp.float32).reshape((1,))
    xt = jnp.transpose(x, (0, 2, 1))
    f32 = jnp.float32
    out = pl.pallas_call(
        _body,
        grid=(_N // _BN,),
        in_specs=[
            pl.BlockSpec(memory_space=pltpu.SMEM),
            pl.BlockSpec((_BN, _F, _S), lambda i: (i, 0, 0)),
        ],
        out_specs=[
            pl.BlockSpec((_BN, _F, _S), lambda i: (i, 0, 0)),
            _dmajor_spec(66), _dmajor_spec(36), _dmajor_spec(21),
            _dmajor_spec(12),
        ],
        out_shape=[
            jax.ShapeDtypeStruct((_N, 96, _S), f32),
            jax.ShapeDtypeStruct((66, _N, _S), f32),
            jax.ShapeDtypeStruct((36, _N, _S), f32),
            jax.ShapeDtypeStruct((21, _N, _S), f32),
            jax.ShapeDtypeStruct((12, _N, _S), f32),
        ],
    )(p, xt)
    x32 = out[0]
    rest = tuple(jnp.transpose(o, (1, 0, 2)) for o in out[1:])
    return (x32,) + rest
